# baseline (speedup 1.0000x reference)
import functools
import math

import jax
import jax.numpy as jnp
from jax import lax
from jax.experimental import pallas as pl
from jax.experimental.pallas import tpu as pltpu
from jax.experimental.pallas import tpu_sc as plsc

_LARGE_NUMBER = 1e10
_NEG = -1e30
_K = 32
_S = 2
_ROWS_PER_BLOCK = 1024
_NW = 32
_RT = _ROWS_PER_BLOCK // _NW


def _laep(x1, x2):
    amax = lax.max(x1, x2)
    delta = lax.sub(x1, x2)
    return lax.add(amax, lax.log1p(lax.exp(lax.neg(lax.abs(delta)))))


def _tc_body(th_ref, marg_ref, q_ref, *, n, kp1):
    f32 = jnp.float32
    neg_row = jnp.full((1, 8, 128), _NEG, f32)
    binit = jnp.concatenate(
        [jnp.zeros((1, 8, 128), f32), jnp.full((kp1 - 1, 8, 128), _NEG, f32)], axis=0)

    def bstep(t, bnext):
        i = n - 1 - t
        th_i = th_ref[pl.ds(i, 1), 0]
        shifted = jnp.concatenate([neg_row, bnext[:-1]], axis=0)
        lognum = th_i + shifted
        bi = _laep(bnext, lognum)
        q_ref[0, pl.ds(i, 1)] = jnp.exp(lognum - bi)[None]
        return bi

    jax.lax.fori_loop(0, n, bstep, binit)

    pi0 = jnp.concatenate(
        [jnp.zeros((kp1 - 1, 8, 128), f32), jnp.ones((1, 8, 128), f32)], axis=0)
    zero_row = jnp.zeros((1, 8, 128), f32)

    def fstep(i, pi_v):
        qi = q_ref[0, pl.ds(i, 1)][0]
        t = pi_v * qi
        marg_ref[pl.ds(i, 1)] = jnp.sum(t, axis=0)[None, None]
        return (pi_v - t) + jnp.concatenate([t[1:], zero_row], axis=0)

    jax.lax.fori_loop(0, n, fstep, pi0)


def _make_sc_sampler(nblocks, n, kp1):
    mesh = plsc.VectorSubcoreMesh(core_axis_name="c", subcore_axis_name="s")
    ngrp = (8 * 128) // 16

    @functools.partial(
        pl.kernel,
        mesh=mesh,
        out_type=jax.ShapeDtypeStruct((nblocks, _S, n, 8, 128), jnp.float32),
        scratch_types=[
            pltpu.VMEM((kp1, 8, 128), jnp.float32),
            pltpu.VMEM((kp1, 8, 128), jnp.float32),
            pltpu.VMEM((_S, 8, 128), jnp.float32),
            pltpu.VMEM((_S, 8, 128), jnp.float32),
            pltpu.VMEM((_S, 8, 128), jnp.float32),
            pltpu.VMEM((_S, ngrp, 16), jnp.int32),
            pltpu.SemaphoreType.DMA,
            pltpu.SemaphoreType.DMA,
        ],
        compiler_params=pltpu.CompilerParams(needs_layout_passes=False),
    )
    def sc_sampler(q_hbm, u_hbm, out_hbm, qv0, qv1, uv0, uv1, mv, rv, semA, semB):
        wid = lax.axis_index("s") * 2 + lax.axis_index("c")
        lane = lax.iota(jnp.int32, 16)

        @pl.when(wid < nblocks)
        def _():
            g = wid

            def step(i, qv, uv):
                for s in range(_S):
                    for lg in range(ngrp):
                        sub = lg // 8
                        lo = 16 * (lg % 8)
                        r = rv[s, lg]
                        qg = plsc.load_gather(
                            qv, [r, jnp.full((16,), sub, jnp.int32), lane + lo])
                        ug = uv[s, sub, pl.ds(lo, 16)]
                        inc = ug < qg
                        mv[s, sub, pl.ds(lo, 16)] = jnp.where(inc, 1.0, 0.0)
                        rv[s, lg] = r - jnp.where(inc, 1, 0)
                pltpu.sync_copy(mv, out_hbm.at[g, :, i])

            for s in range(_S):
                for lg in range(ngrp):
                    rv[s, lg] = jnp.full((16,), _K, jnp.int32)

            pltpu.sync_copy(q_hbm.at[g, 0], qv0)
            pltpu.sync_copy(u_hbm.at[g, 0], uv0)

            def chunk(kk, _):
                i0 = 2 * kk
                i1 = 2 * kk + 1
                i2 = jnp.minimum(2 * kk + 2, n - 1)
                cp_q1 = pltpu.make_async_copy(q_hbm.at[g, i1], qv1, semA)
                cp_u1 = pltpu.make_async_copy(u_hbm.at[g, i1], uv1, semA)
                cp_q1.start()
                cp_u1.start()
                step(i0, qv0, uv0)
                cp_q1.wait()
                cp_u1.wait()
                cp_q0 = pltpu.make_async_copy(q_hbm.at[g, i2], qv0, semB)
                cp_u0 = pltpu.make_async_copy(u_hbm.at[g, i2], uv0, semB)
                cp_q0.start()
                cp_u0.start()
                step(i1, qv1, uv1)
                cp_q0.wait()
                cp_u0.wait()
                return 0

            lax.fori_loop(0, n // 2, chunk, 0)

    return sc_sampler


def kernel(scores):
    nnodes, choices, ensemble = scores.shape
    local_k = min(_K, choices)
    kp1 = local_k + 1
    n = 2 ** int(math.ceil(math.log2(choices)))
    rows = nnodes * ensemble
    rpb = _ROWS_PER_BLOCK
    nblocks = (rows + rpb - 1) // rpb
    rows_pad = nblocks * rpb

    th = jnp.transpose(scores, (1, 0, 2)).reshape(choices, rows)
    if n > choices:
        th = jnp.concatenate(
            [th, jnp.full((n - choices, rows), -_LARGE_NUMBER, th.dtype)], axis=0)
    th4 = jnp.pad(th, ((0, 0), (0, rows_pad - rows))).reshape(n, nblocks, 8, 128)

    body = functools.partial(_tc_body, n=n, kp1=kp1)
    marg4, q5 = pl.pallas_call(
        body,
        grid=(nblocks,),
        in_specs=[pl.BlockSpec((n, 1, 8, 128), lambda g: (0, g, 0, 0))],
        out_specs=[
            pl.BlockSpec((n, 1, 8, 128), lambda g: (0, g, 0, 0)),
            pl.BlockSpec((1, n, kp1, 8, 128), lambda g: (g, 0, 0, 0, 0)),
        ],
        out_shape=[
            jax.ShapeDtypeStruct((n, nblocks, 8, 128), jnp.float32),
            jax.ShapeDtypeStruct((nblocks, n, kp1, 8, 128), jnp.float32),
        ],
    )(th4)

    u = jax.random.uniform(jax.random.key(1), (n, _S, rows), dtype=scores.dtype)
    u2 = jnp.pad(u.reshape(n * _S, rows), ((0, 0), (0, rows_pad - rows)))
    u5 = u2.reshape(n, _S, nblocks, 8, 128).transpose(2, 0, 1, 3, 4)

    masks5 = _make_sc_sampler(nblocks, n, kp1)(q5, u5)

    marg = marg4.reshape(n, rows_pad)[:choices, :rows]
    marginals = marg.reshape(choices, nnodes, ensemble).transpose(1, 0, 2)

    masks = masks5.transpose(1, 2, 0, 3, 4).reshape(_S, n, rows_pad)[:, :choices, :rows]
    sb = masks.reshape(_S, choices, nnodes, ensemble).transpose(0, 2, 1, 3)
    samples = jax.lax.stop_gradient(sb - marginals[None]) + marginals[None]
    return samples, marginals


# speedup vs baseline: 1.7514x; 1.7514x over previous
import functools
import math

import jax
import jax.numpy as jnp
from jax import lax
from jax.experimental import pallas as pl
from jax.experimental.pallas import tpu as pltpu

_LARGE_NUMBER = 1e10
_NEG = -1e30
_K = 32
_S = 2
_ROWS_PER_BLOCK = 1024


def _laep(x1, x2):
    amax = lax.max(x1, x2)
    delta = lax.sub(x1, x2)
    return lax.add(amax, lax.log1p(lax.exp(lax.neg(lax.abs(delta)))))


def _simple_body(th_ref, u_ref, marg_ref, masks_ref, qscr, *, n, kp1):
    f32 = jnp.float32
    neg_row = jnp.full((1, 8, 128), _NEG, f32)

    b = jnp.zeros((1, 8, 128), f32)
    for t in range(kp1 - 2):
        i = n - 1 - t
        th_i = th_ref[pl.ds(i, 1), 0]
        shifted = jnp.concatenate([neg_row, b], axis=0)
        bnext_ext = jnp.concatenate([b, neg_row], axis=0)
        lognum = th_i + shifted
        b = _laep(bnext_ext, lognum)
        qscr[i, : t + 2] = jnp.exp(lognum - b)

    binit = jnp.concatenate([b, jnp.full((kp1 - b.shape[0], 8, 128), _NEG, f32)], axis=0)

    def bstep(t, bnext):
        i = n - 1 - t
        th_i = th_ref[pl.ds(i, 1), 0]
        shifted = jnp.concatenate([neg_row, bnext[:-1]], axis=0)
        lognum = th_i + shifted
        bi = _laep(bnext, lognum)
        qscr[pl.ds(i, 1)] = jnp.exp(lognum - bi)[None]
        return bi

    jax.lax.fori_loop(kp1 - 2, n, bstep, binit)

    zero_row = jnp.zeros((1, 8, 128), f32)
    pi_v = jnp.ones((1, 8, 128), f32)
    rs = [jnp.full((8, 128), _K, jnp.int32) for _ in range(_S)]
    for i in range(n):
        lo = max(0, _K - i)
        hi = min(_K, n - i)
        w = hi - lo + 1
        qi = qscr[i, lo : hi + 1]
        t = pi_v * qi
        marg_ref[pl.ds(i, 1)] = jnp.sum(t, axis=0)[None, None]
        base = pi_v - t
        tup = zero_row if w == 1 else jnp.concatenate([t[1:], zero_row], axis=0)
        lo2 = max(0, _K - i - 1)
        hi2 = min(_K, n - i - 1)
        if lo2 < lo:
            pi_v = jnp.concatenate([t[:1], base + tup], axis=0)
        else:
            pi_v = (base + tup)[: hi2 - lo2 + 1]
        jjw = lax.broadcasted_iota(jnp.int32, (w, 8, 128), 0) + lo
        for s in range(_S):
            r = rs[s]
            p = jnp.sum(jnp.where(jjw == r[None], qi, 0.0), axis=0)
            u = u_ref[pl.ds(_S * i + s, 1), 0][0]
            inc = u < p
            masks_ref[pl.ds(s * n + i, 1)] = inc.astype(f32)[None, None]
            rs[s] = r - inc.astype(jnp.int32)


def kernel(scores):
    nnodes, choices, ensemble = scores.shape
    local_k = min(_K, choices)
    kp1 = local_k + 1
    n = 2 ** int(math.ceil(math.log2(choices)))
    rows = nnodes * ensemble
    rpb = _ROWS_PER_BLOCK
    nblocks = (rows + rpb - 1) // rpb
    rows_pad = nblocks * rpb

    th = jnp.transpose(scores, (1, 0, 2)).reshape(choices, rows)
    if n > choices:
        th = jnp.concatenate(
            [th, jnp.full((n - choices, rows), -_LARGE_NUMBER, th.dtype)], axis=0)
    th4 = jnp.pad(th, ((0, 0), (0, rows_pad - rows))).reshape(n, nblocks, 8, 128)

    u = jax.random.uniform(jax.random.key(1), (n, _S, rows), dtype=scores.dtype)
    u4 = jnp.pad(u.reshape(n * _S, rows), ((0, 0), (0, rows_pad - rows)))
    u4 = u4.reshape(n * _S, nblocks, 8, 128)

    body = functools.partial(_simple_body, n=n, kp1=kp1)
    marg4, masks4 = pl.pallas_call(
        body,
        grid=(nblocks,),
        in_specs=[
            pl.BlockSpec((n, 1, 8, 128), lambda g: (0, g, 0, 0)),
            pl.BlockSpec((n * _S, 1, 8, 128), lambda g: (0, g, 0, 0)),
        ],
        out_specs=[
            pl.BlockSpec((n, 1, 8, 128), lambda g: (0, g, 0, 0)),
            pl.BlockSpec((_S * n, 1, 8, 128), lambda g: (0, g, 0, 0)),
        ],
        out_shape=[
            jax.ShapeDtypeStruct((n, nblocks, 8, 128), jnp.float32),
            jax.ShapeDtypeStruct((_S * n, nblocks, 8, 128), jnp.float32),
        ],
        scratch_shapes=[
            pltpu.VMEM((n, kp1, 8, 128), jnp.float32),
        ],
    )(th4, u4)

    marg = marg4.reshape(n, rows_pad)[:choices, :rows]
    marginals = marg.reshape(choices, nnodes, ensemble).transpose(1, 0, 2)

    masks = masks4.reshape(_S, n, rows_pad)[:, :choices, :rows]
    sb = masks.reshape(_S, choices, nnodes, ensemble).transpose(0, 2, 1, 3)
    samples = jax.lax.stop_gradient(sb - marginals[None]) + marginals[None]
    return samples, marginals


# speedup vs baseline: 1.9884x; 1.1353x over previous
import functools
import math

import jax
import jax.numpy as jnp
from jax import lax
from jax.experimental import pallas as pl
from jax.experimental.pallas import tpu as pltpu

_LARGE_NUMBER = 1e10
_NEG = -1e30
_K = 32
_S = 2
_ROWS_PER_BLOCK = 1024


def _laep(x1, x2):
    amax = lax.max(x1, x2)
    delta = lax.sub(x1, x2)
    return lax.add(amax, lax.log1p(lax.exp(lax.neg(lax.abs(delta)))))


def _simple_body(th_ref, u_ref, marg_ref, masks_ref, qscr, *, n, kp1):
    f32 = jnp.float32
    neg_row = jnp.full((1, 8, 128), _NEG, f32)

    b = jnp.zeros((1, 8, 128), f32)
    for i in range(n - 1, _K - 1, -1):
        th_i = th_ref[pl.ds(i, 1), 0]
        shifted = jnp.concatenate([neg_row, b], axis=0)
        bnext_ext = jnp.concatenate([b, neg_row], axis=0)
        lognum = th_i + shifted
        b = _laep(bnext_ext, lognum)
        qscr[i, : b.shape[0]] = jnp.exp(lognum - b)

    for i in range(_K - 1, -1, -1):
        th_i = th_ref[pl.ds(i, 1), 0]
        lognum = th_i + b[:-1]
        bnew = _laep(b[1:], lognum)
        qscr[i, _K - i : _K + 1] = jnp.exp(lognum - bnew)
        b = bnew

    zero_row = jnp.zeros((1, 8, 128), f32)
    pi_v = jnp.ones((1, 8, 128), f32)
    rs = [jnp.full((8, 128), _K, jnp.int32) for _ in range(_S)]
    for i in range(n):
        lo = max(0, _K - i)
        hi = min(_K, n - i)
        w = hi - lo + 1
        qi = qscr[i, lo : hi + 1]
        t = pi_v * qi
        marg_ref[pl.ds(i, 1)] = jnp.sum(t, axis=0)[None, None]
        base = pi_v - t
        tup = zero_row if w == 1 else jnp.concatenate([t[1:], zero_row], axis=0)
        lo2 = max(0, _K - i - 1)
        hi2 = min(_K, n - i - 1)
        if lo2 < lo:
            pi_v = jnp.concatenate([t[:1], base + tup], axis=0)
        else:
            pi_v = (base + tup)[: hi2 - lo2 + 1]
        jjw = lax.broadcasted_iota(jnp.int32, (w, 8, 128), 0) + lo
        for s in range(_S):
            r = rs[s]
            p = jnp.sum(jnp.where(jjw == r[None], qi, 0.0), axis=0)
            u = u_ref[pl.ds(_S * i + s, 1), 0][0]
            inc = u < p
            masks_ref[pl.ds(s * n + i, 1)] = inc.astype(f32)[None, None]
            rs[s] = r - inc.astype(jnp.int32)


def kernel(scores):
    nnodes, choices, ensemble = scores.shape
    local_k = min(_K, choices)
    kp1 = local_k + 1
    n = 2 ** int(math.ceil(math.log2(choices)))
    rows = nnodes * ensemble
    rpb = _ROWS_PER_BLOCK
    nblocks = (rows + rpb - 1) // rpb
    rows_pad = nblocks * rpb

    th = jnp.transpose(scores, (1, 0, 2)).reshape(choices, rows)
    if n > choices:
        th = jnp.concatenate(
            [th, jnp.full((n - choices, rows), -_LARGE_NUMBER, th.dtype)], axis=0)
    th4 = jnp.pad(th, ((0, 0), (0, rows_pad - rows))).reshape(n, nblocks, 8, 128)

    u = jax.random.uniform(jax.random.key(1), (n, _S, rows), dtype=scores.dtype)
    u4 = jnp.pad(u.reshape(n * _S, rows), ((0, 0), (0, rows_pad - rows)))
    u4 = u4.reshape(n * _S, nblocks, 8, 128)

    body = functools.partial(_simple_body, n=n, kp1=kp1)
    marg4, masks4 = pl.pallas_call(
        body,
        grid=(nblocks,),
        in_specs=[
            pl.BlockSpec((n, 1, 8, 128), lambda g: (0, g, 0, 0)),
            pl.BlockSpec((n * _S, 1, 8, 128), lambda g: (0, g, 0, 0)),
        ],
        out_specs=[
            pl.BlockSpec((n, 1, 8, 128), lambda g: (0, g, 0, 0)),
            pl.BlockSpec((_S * n, 1, 8, 128), lambda g: (0, g, 0, 0)),
        ],
        out_shape=[
            jax.ShapeDtypeStruct((n, nblocks, 8, 128), jnp.float32),
            jax.ShapeDtypeStruct((_S * n, nblocks, 8, 128), jnp.float32),
        ],
        scratch_shapes=[
            pltpu.VMEM((n, kp1, 8, 128), jnp.float32),
        ],
    )(th4, u4)

    marg = marg4.reshape(n, rows_pad)[:choices, :rows]
    marginals = marg.reshape(choices, nnodes, ensemble).transpose(1, 0, 2)

    masks = masks4.reshape(_S, n, rows_pad)[:, :choices, :rows]
    sb = masks.reshape(_S, choices, nnodes, ensemble).transpose(0, 2, 1, 3)
    samples = jax.lax.stop_gradient(sb - marginals[None]) + marginals[None]
    return samples, marginals


# speedup vs baseline: 2.1593x; 1.0859x over previous
import functools
import math

import jax
import jax.numpy as jnp
from jax import lax
from jax.experimental import pallas as pl
from jax.experimental.pallas import tpu as pltpu

_LARGE_NUMBER = 1e10
_NEG = -1e30
_K = 32
_S = 2
_ROWS_PER_BLOCK = 1024


def _laep(x1, x2):
    amax = lax.max(x1, x2)
    delta = lax.sub(x1, x2)
    return lax.add(amax, lax.log1p(lax.exp(lax.neg(lax.abs(delta)))))


def _simple_body(th_ref, u_ref, marg_ref, masks_ref, qscr, *, n, kp1):
    f32 = jnp.float32
    neg_row = jnp.full((1, 8, 128), _NEG, f32)

    b = jnp.zeros((1, 8, 128), f32)
    for i in range(n - 1, _K - 1, -1):
        th_i = th_ref[pl.ds(i, 1), 0]
        shifted = jnp.concatenate([neg_row, b], axis=0)
        bnext_ext = jnp.concatenate([b, neg_row], axis=0)
        lognum = th_i + shifted
        b = _laep(bnext_ext, lognum)
        qscr[i, : b.shape[0]] = jnp.exp(lognum - b)

    for i in range(_K - 1, -1, -1):
        th_i = th_ref[pl.ds(i, 1), 0]
        lognum = th_i + b[:-1]
        bnew = _laep(b[1:], lognum)
        qscr[i, _K - i : _K + 1] = jnp.exp(lognum - bnew)
        b = bnew

    zero_row = jnp.zeros((1, 8, 128), f32)
    pi_v = jnp.ones((1, 8, 128), f32)
    rs = [jnp.full((8, 128), _K, jnp.int32) for _ in range(_S)]
    for i in range(n):
        lo = max(0, _K - i)
        hi = min(_K, n - i)
        w = hi - lo + 1
        qi = qscr[i, lo : hi + 1]
        t = pi_v * qi
        marg_i = jnp.sum(t, axis=0)
        marg_ref[pl.ds(i, 1)] = marg_i[None, None]
        base = pi_v - t
        tup = zero_row if w == 1 else jnp.concatenate([t[1:], zero_row], axis=0)
        lo2 = max(0, _K - i - 1)
        hi2 = min(_K, n - i - 1)
        if lo2 < lo:
            pi_v = jnp.concatenate([t[:1], base + tup], axis=0)
        else:
            pi_v = (base + tup)[: hi2 - lo2 + 1]
        for s in range(_S):
            r = rs[s]
            x = r - lo
            vals = [qi[j] for j in range(w)]
            level = 0
            while len(vals) > 1:
                bit = (x & (1 << level)) != 0
                vals = [
                    jnp.where(bit, vals[2 * m + 1], vals[2 * m])
                    if 2 * m + 1 < len(vals) else vals[2 * m]
                    for m in range((len(vals) + 1) // 2)
                ]
                level += 1
            p = vals[0]
            u = u_ref[pl.ds(_S * i + s, 1), 0][0]
            inc = u < p
            st = (inc.astype(f32) - marg_i) + marg_i
            masks_ref[pl.ds(s * n + i, 1)] = st[None, None]
            rs[s] = r - inc.astype(jnp.int32)


def kernel(scores):
    nnodes, choices, ensemble = scores.shape
    local_k = min(_K, choices)
    kp1 = local_k + 1
    n = 2 ** int(math.ceil(math.log2(choices)))
    rows = nnodes * ensemble
    rpb = _ROWS_PER_BLOCK
    nblocks = (rows + rpb - 1) // rpb
    rows_pad = nblocks * rpb

    th = jnp.transpose(scores, (1, 0, 2)).reshape(choices, rows)
    if n > choices:
        th = jnp.concatenate(
            [th, jnp.full((n - choices, rows), -_LARGE_NUMBER, th.dtype)], axis=0)
    th4 = jnp.pad(th, ((0, 0), (0, rows_pad - rows))).reshape(n, nblocks, 8, 128)

    u = jax.random.uniform(jax.random.key(1), (n, _S, rows), dtype=scores.dtype)
    u4 = jnp.pad(u.reshape(n * _S, rows), ((0, 0), (0, rows_pad - rows)))
    u4 = u4.reshape(n * _S, nblocks, 8, 128)

    body = functools.partial(_simple_body, n=n, kp1=kp1)
    marg4, masks4 = pl.pallas_call(
        body,
        grid=(nblocks,),
        in_specs=[
            pl.BlockSpec((n, 1, 8, 128), lambda g: (0, g, 0, 0)),
            pl.BlockSpec((n * _S, 1, 8, 128), lambda g: (0, g, 0, 0)),
        ],
        out_specs=[
            pl.BlockSpec((n, 1, 8, 128), lambda g: (0, g, 0, 0)),
            pl.BlockSpec((_S * n, 1, 8, 128), lambda g: (0, g, 0, 0)),
        ],
        out_shape=[
            jax.ShapeDtypeStruct((n, nblocks, 8, 128), jnp.float32),
            jax.ShapeDtypeStruct((_S * n, nblocks, 8, 128), jnp.float32),
        ],
        scratch_shapes=[
            pltpu.VMEM((n, kp1, 8, 128), jnp.float32),
        ],
    )(th4, u4)

    marg = marg4.reshape(n, rows_pad)[:choices, :rows]
    marginals = marg.reshape(choices, nnodes, ensemble).transpose(1, 0, 2)

    masks = masks4.reshape(_S, n, rows_pad)[:, :choices, :rows]
    samples = masks.reshape(_S, choices, nnodes, ensemble).transpose(0, 2, 1, 3)
    return samples, marginals
